# Initial kernel scaffold; baseline (speedup 1.0000x reference)
import jax
import jax.numpy as jnp
from jax.experimental import pallas as pl


def kernel(img_lr, h_weight, d_weight, v_weight):
    raise NotImplementedError("write your pallas kernel here")


# speedup vs baseline: 292.5973x; 292.5973x over previous
import functools

import jax
import jax.numpy as jnp
from jax import lax
from jax.experimental import pallas as pl
from jax.experimental.pallas import tpu as pltpu
from jax.experimental.pallas import tpu_sc as plsc

L = 256
NUM_WORKERS = 32

_PERM = {
    0: (0, 1, 2, 3),
    1: (1, 3, 0, 2),
    2: (3, 2, 1, 0),
    3: (2, 0, 3, 1),
}

_DIRS = (
    (0, 0, 'c', +1),
    (0, 2, 'c', -1),
    (1, 0, 'n', 0),
    (1, 2, 'p', 0),
    (2, 0, 'n', +1),
    (2, 3, 'n', -1),
    (2, 2, 'p', -1),
    (2, 1, 'p', +1),
)


def _sc_body(H, W, rows_per_worker, img, tabcat, out,
             tabs, p0, p8, c0, c8, n0, n8,
             idxs, stages, a256, outt, outb, sem):
    ng_idx = W // 16

    sid = lax.axis_index("s")

    @pl.when(sid == 0)
    def _stage_tables():
        for k in range(12):
            pltpu.sync_copy(tabcat.at[pl.ds(k * L * L, L * L)], tabs[k])

    plsc.subcore_barrier()

    wid = lax.axis_index("s") * 2 + lax.axis_index("c")
    row0 = wid * rows_per_worker

    io16 = lax.iota(jnp.int32, 16)
    mask_last = io16 == 15
    mask_first = io16 == 0
    splat_w = jnp.full((16,), W, jnp.int32)
    splat_7 = jnp.full((16,), 7, jnp.int32)
    sc_even = io16 * 2

    def row_body(i, carry):
        r = row0 + i
        plane = lax.div(r, H)
        orow = r - plane * H
        prev_r = jnp.where(orow == 0, r, r - 1)
        next_r = jnp.where(orow == H - 1, r, r + 1)

        pltpu.sync_copy(img.at[pl.ds(r * W, W)], c0.at[pl.ds(0, W)])
        pltpu.sync_copy(img.at[pl.ds(r * W, W)], c8.at[pl.ds(8, W)])
        pltpu.sync_copy(img.at[pl.ds(prev_r * W, W)], p0.at[pl.ds(0, W)])
        pltpu.sync_copy(img.at[pl.ds(prev_r * W, W)], p8.at[pl.ds(8, W)])
        pltpu.sync_copy(img.at[pl.ds(next_r * W, W)], n0.at[pl.ds(0, W)])
        pltpu.sync_copy(img.at[pl.ds(next_r * W, W)], n8.at[pl.ds(8, W)])

        for buf in (c0, p0, n0):
            v = buf[pl.ds(W - 16, 16)]
            plsc.store_scatter(buf, [splat_w], v, mask=mask_last)
        for buf in (c8, p8, n8):
            v = buf[pl.ds(8, 16)]
            plsc.store_scatter(buf, [splat_7], v, mask=mask_first)

        def a_body(g, carry):
            a = c0[pl.ds(g * 16, 16)]
            a256[pl.ds(g * 16, 16)] = a * 256.0
            return carry
        lax.fori_loop(0, ng_idx, a_body, 0, unroll=2)

        handles = []
        for d, (t, _, rs, dj) in enumerate(_DIRS):
            if dj >= 0:
                buf = {'c': c0, 'p': p0, 'n': n0}[rs]
                off = dj
            else:
                buf = {'c': c8, 'p': p8, 'n': n8}[rs]
                off = 7
            idx_ref = idxs[d]

            def i_body(g, carry, buf=buf, off=off, idx_ref=idx_ref):
                av = a256[pl.ds(g * 16, 16)]
                bv = buf[pl.ds(g * 16 + off, 16)]
                idx_ref[pl.ds(g * 16, 16)] = (av + bv).astype(jnp.int32)
                return carry
            lax.fori_loop(0, ng_idx, i_body, 0, unroll=2)
            for c in range(4):
                handles.append(pltpu.async_copy(
                    tabs[4 * t + c].at[idx_ref], stages[4 * d + c], sem))
        for h in handles:
            h.wait()

        def o_body(g, carry):
            te = to = be = bo = None
            for d in range(8):
                k = _DIRS[d][1]
                pm = _PERM[k]
                sl = pl.ds(g * 16, 16)
                vte = stages[4 * d + pm[0]][sl]
                vto = stages[4 * d + pm[1]][sl]
                vbe = stages[4 * d + pm[2]][sl]
                vbo = stages[4 * d + pm[3]][sl]
                te = vte if te is None else te + vte
                to = vto if to is None else to + vto
                be = vbe if be is None else be + vbe
                bo = vbo if bo is None else bo + vbo
            sc = sc_even + g * 32
            plsc.store_scatter(outt, [sc], te * 0.5)
            plsc.store_scatter(outt, [sc + 1], to * 0.5)
            plsc.store_scatter(outb, [sc], be * 0.5)
            plsc.store_scatter(outb, [sc + 1], bo * 0.5)
            return carry
        lax.fori_loop(0, ng_idx, o_body, 0, unroll=2)

        obase = plane * (2 * H * 2 * W) + orow * (4 * W)
        pltpu.sync_copy(outt, out.at[pl.ds(obase, 2 * W)])
        pltpu.sync_copy(outb, out.at[pl.ds(obase + 2 * W, 2 * W)])
        return carry

    lax.fori_loop(0, rows_per_worker, row_body, 0)


@functools.partial(jax.jit, static_argnums=(2, 3, 4, 5))
def _sc_call(img_flat, tabcat, nrows, W, H, interpret=False):
    rows_per_worker = nrows // NUM_WORKERS
    mesh = plsc.VectorSubcoreMesh(core_axis_name="c", subcore_axis_name="s",
                                  num_cores=2, num_subcores=16)
    body = functools.partial(_sc_body, H, W, rows_per_worker)
    return pl.kernel(
        body,
        out_type=jax.ShapeDtypeStruct((nrows * 4 * W,), jnp.float32),
        mesh=mesh,
        interpret=interpret,
        compiler_params=pltpu.CompilerParams(needs_layout_passes=False,
                                             use_tc_tiling_on_sc=False),
        scratch_types=dict(
            tabs=[pltpu.VMEM_SHARED((L * L,), jnp.float32)
                  for _ in range(12)],
            p0=pltpu.VMEM((W + 16,), jnp.float32),
            p8=pltpu.VMEM((W + 16,), jnp.float32),
            c0=pltpu.VMEM((W + 16,), jnp.float32),
            c8=pltpu.VMEM((W + 16,), jnp.float32),
            n0=pltpu.VMEM((W + 16,), jnp.float32),
            n8=pltpu.VMEM((W + 16,), jnp.float32),
            idxs=[pltpu.VMEM((W,), jnp.int32) for _ in range(8)],
            stages=[pltpu.VMEM((W,), jnp.float32) for _ in range(32)],
            a256=pltpu.VMEM((W,), jnp.float32),
            outt=pltpu.VMEM((2 * W,), jnp.float32),
            outb=pltpu.VMEM((2 * W,), jnp.float32),
            sem=pltpu.SemaphoreType.DMA,
        ),
    )(img_flat, tabcat)


def kernel(img_lr, h_weight, d_weight, v_weight):
    B, C, H, W = img_lr.shape
    img = img_lr.reshape(-1)
    tabcat = jnp.concatenate(
        [w.reshape(L * L, 4).T.reshape(-1)
         for w in (h_weight, v_weight, d_weight)])
    out = _sc_call(img, tabcat, B * C * H, W, H)
    return out.reshape(B, C, 2 * H, 2 * W)


# speedup vs baseline: 406.1672x; 1.3881x over previous
import functools

import jax
import jax.numpy as jnp
from jax import lax
from jax.experimental import pallas as pl
from jax.experimental.pallas import tpu as pltpu
from jax.experimental.pallas import tpu_sc as plsc

L = 256
NUM_WORKERS = 32

_PERM = {
    0: (0, 1, 2, 3),
    1: (1, 3, 0, 2),
    2: (3, 2, 1, 0),
    3: (2, 0, 3, 1),
}

_DIRS = (
    (0, 0, 'c', +1),
    (0, 2, 'c', -1),
    (1, 0, 'n', 0),
    (1, 2, 'p', 0),
    (2, 0, 'n', +1),
    (2, 3, 'n', -1),
    (2, 2, 'p', -1),
    (2, 1, 'p', +1),
)


def _sc_body(H, W, rows_per_worker, img, tabcat, out,
             tabs, p0, p8, c0, c8, n0, n8,
             idxs, stages, a256, outt, outb, sem):
    ng_idx = W // 16

    sid = lax.axis_index("s")

    @pl.when(sid == 0)
    def _stage_tables():
        for k in range(6):
            pltpu.sync_copy(tabcat.at[pl.ds(k * L * L, L * L)], tabs[k])

    plsc.subcore_barrier()

    wid = lax.axis_index("s") * 2 + lax.axis_index("c")
    row0 = wid * rows_per_worker

    io16 = lax.iota(jnp.int32, 16)
    mask_last = io16 == 15
    mask_first = io16 == 0
    splat_w = jnp.full((16,), W, jnp.int32)
    splat_7 = jnp.full((16,), 7, jnp.int32)
    sc_even = io16 * 2

    def row_body(i, carry):
        r = row0 + i
        plane = lax.div(r, H)
        orow = r - plane * H
        prev_r = jnp.where(orow == 0, r, r - 1)
        next_r = jnp.where(orow == H - 1, r, r + 1)

        pltpu.sync_copy(img.at[pl.ds(r * W, W)], c0.at[pl.ds(0, W)])
        pltpu.sync_copy(img.at[pl.ds(r * W, W)], c8.at[pl.ds(8, W)])
        pltpu.sync_copy(img.at[pl.ds(prev_r * W, W)], p0.at[pl.ds(0, W)])
        pltpu.sync_copy(img.at[pl.ds(prev_r * W, W)], p8.at[pl.ds(8, W)])
        pltpu.sync_copy(img.at[pl.ds(next_r * W, W)], n0.at[pl.ds(0, W)])
        pltpu.sync_copy(img.at[pl.ds(next_r * W, W)], n8.at[pl.ds(8, W)])

        for buf in (c0, p0, n0):
            v = buf[pl.ds(W - 16, 16)]
            plsc.store_scatter(buf, [splat_w], v, mask=mask_last)
        for buf in (c8, p8, n8):
            v = buf[pl.ds(8, 16)]
            plsc.store_scatter(buf, [splat_7], v, mask=mask_first)

        def a_body(g, carry):
            a = c0[pl.ds(g * 16, 16)]
            a256[pl.ds(g * 16, 16)] = a * 256.0
            return carry
        lax.fori_loop(0, ng_idx, a_body, 0, unroll=2)

        handles = []
        for d, (t, _, rs, dj) in enumerate(_DIRS):
            if dj >= 0:
                buf = {'c': c0, 'p': p0, 'n': n0}[rs]
                off = dj
            else:
                buf = {'c': c8, 'p': p8, 'n': n8}[rs]
                off = 7
            idx_ref = idxs[d]

            def i_body(g, carry, buf=buf, off=off, idx_ref=idx_ref):
                av = a256[pl.ds(g * 16, 16)]
                bv = buf[pl.ds(g * 16 + off, 16)]
                idx_ref[pl.ds(g * 16, 16)] = (av + bv).astype(jnp.int32)
                return carry
            lax.fori_loop(0, ng_idx, i_body, 0, unroll=2)
            for c in range(2):
                handles.append(pltpu.async_copy(
                    tabs[2 * t + c].at[idx_ref], stages[2 * d + c], sem))
        for h in handles:
            h.wait()

        himask = jnp.full((16,), -65536, jnp.int32)

        def o_body(g, carry):
            sl = pl.ds(g * 16, 16)
            te = to = be = bo = None

            def ext(w, half):
                bits = lax.shift_left(w, jnp.full((16,), 16, jnp.int32)) \
                    if half == 0 else (w & himask)
                return plsc.bitcast(bits, jnp.float32)

            for d in range(8):
                k = _DIRS[d][1]
                pm = _PERM[k]
                wt = stages[2 * d][sl]
                wb = stages[2 * d + 1][sl]
                vals = [ext(wt if p < 2 else wb, p % 2) for p in pm]
                te = vals[0] if te is None else te + vals[0]
                to = vals[1] if to is None else to + vals[1]
                be = vals[2] if be is None else be + vals[2]
                bo = vals[3] if bo is None else bo + vals[3]
            sc = sc_even + g * 32
            plsc.store_scatter(outt, [sc], te)
            plsc.store_scatter(outt, [sc + 1], to)
            plsc.store_scatter(outb, [sc], be)
            plsc.store_scatter(outb, [sc + 1], bo)
            return carry
        lax.fori_loop(0, ng_idx, o_body, 0, unroll=2)

        obase = plane * (2 * H * 2 * W) + orow * (4 * W)
        pltpu.sync_copy(outt, out.at[pl.ds(obase, 2 * W)])
        pltpu.sync_copy(outb, out.at[pl.ds(obase + 2 * W, 2 * W)])
        return carry

    lax.fori_loop(0, rows_per_worker, row_body, 0)


@functools.partial(jax.jit, static_argnums=(2, 3, 4, 5))
def _sc_call(img_flat, tabcat, nrows, W, H, interpret=False):
    rows_per_worker = nrows // NUM_WORKERS
    mesh = plsc.VectorSubcoreMesh(core_axis_name="c", subcore_axis_name="s",
                                  num_cores=2, num_subcores=16)
    body = functools.partial(_sc_body, H, W, rows_per_worker)
    return pl.kernel(
        body,
        out_type=jax.ShapeDtypeStruct((nrows * 4 * W,), jnp.float32),
        mesh=mesh,
        interpret=interpret,
        compiler_params=pltpu.CompilerParams(needs_layout_passes=False,
                                             use_tc_tiling_on_sc=False),
        scratch_types=dict(
            tabs=[pltpu.VMEM_SHARED((L * L,), jnp.int32)
                  for _ in range(6)],
            p0=pltpu.VMEM((W + 16,), jnp.float32),
            p8=pltpu.VMEM((W + 16,), jnp.float32),
            c0=pltpu.VMEM((W + 16,), jnp.float32),
            c8=pltpu.VMEM((W + 16,), jnp.float32),
            n0=pltpu.VMEM((W + 16,), jnp.float32),
            n8=pltpu.VMEM((W + 16,), jnp.float32),
            idxs=[pltpu.VMEM((W,), jnp.int32) for _ in range(8)],
            stages=[pltpu.VMEM((W,), jnp.int32) for _ in range(16)],
            a256=pltpu.VMEM((W,), jnp.float32),
            outt=pltpu.VMEM((2 * W,), jnp.float32),
            outb=pltpu.VMEM((2 * W,), jnp.float32),
            sem=pltpu.SemaphoreType.DMA,
        ),
    )(img_flat, tabcat)


def kernel(img_lr, h_weight, d_weight, v_weight):
    B, C, H, W = img_lr.shape
    img = img_lr.reshape(-1)

    def pack_pair(lo, hi):
        lob = jax.lax.bitcast_convert_type(
            (0.5 * lo).astype(jnp.bfloat16), jnp.uint16).astype(jnp.uint32)
        hib = jax.lax.bitcast_convert_type(
            (0.5 * hi).astype(jnp.bfloat16), jnp.uint16).astype(jnp.uint32)
        return ((hib << 16) | lob).astype(jnp.int32)

    planes = []
    for w in (h_weight, v_weight, d_weight):
        wf = w.reshape(L * L, 4)
        planes.append(pack_pair(wf[:, 0], wf[:, 1]))
        planes.append(pack_pair(wf[:, 2], wf[:, 3]))
    tabcat = jnp.concatenate(planes)
    out = _sc_call(img, tabcat, B * C * H, W, H)
    return out.reshape(B, C, 2 * H, 2 * W)


# speedup vs baseline: 825.9334x; 2.0335x over previous
import functools

import jax
import jax.numpy as jnp
from jax import lax
from jax.experimental import pallas as pl
from jax.experimental.pallas import tpu as pltpu
from jax.experimental.pallas import tpu_sc as plsc

L = 256
NUM_WORKERS = 32

_PERM = {
    0: (0, 1, 2, 3),
    1: (1, 3, 0, 2),
    2: (3, 2, 1, 0),
    3: (2, 0, 3, 1),
}

_DIRS = (
    (0, 0, 'c', +1),
    (0, 2, 'c', -1),
    (1, 0, 'n', 0),
    (1, 2, 'p', 0),
    (2, 0, 'n', +1),
    (2, 3, 'n', -1),
    (2, 2, 'p', -1),
    (2, 1, 'p', +1),
)


def _sc_body(H, W, rows_per_worker, img, tabcat, out,
             tabs, rowbufs, idxs, stages, a256, outts, outbs,
             semr, semg, semo):
    ng_idx = W // 16

    sid = lax.axis_index("s")

    @pl.when(sid == 0)
    def _stage_tables():
        for k in range(6):
            pltpu.sync_copy(tabcat.at[pl.ds(k * L * L, L * L)], tabs[k])

    plsc.subcore_barrier()

    wid = lax.axis_index("s") * 2 + lax.axis_index("c")
    row0 = wid * rows_per_worker
    rlast = row0 + rows_per_worker - 1

    io16 = lax.iota(jnp.int32, 16)
    mask_last = io16 == 15
    mask_first = io16 == 0
    splat_w = jnp.full((16,), W, jnp.int32)
    splat_7 = jnp.full((16,), 7, jnp.int32)
    sc_even = io16 * 2
    himask = jnp.full((16,), -65536, jnp.int32)
    sh16 = jnp.full((16,), 16, jnp.int32)


    def load_rows(r, S):
        r = jnp.minimum(r, rlast)
        plane = lax.div(r, H)
        orow = r - plane * H
        prev_r = jnp.where(orow == 0, r, r - 1)
        next_r = jnp.where(orow == H - 1, r, r + 1)
        c0, c8, p0, p8, n0, n8 = rowbufs[S]
        for src_r, (b0, b8) in ((r, (c0, c8)), (prev_r, (p0, p8)),
                                (next_r, (n0, n8))):
            pltpu.async_copy(img.at[pl.ds(src_r * W, W)],
                             b0.at[pl.ds(0, W)], semr[S])
            pltpu.async_copy(img.at[pl.ds(src_r * W, W)],
                             b8.at[pl.ds(8, W)], semr[S])

    def wait_rows(S):
        for b in rowbufs[S]:
            pltpu.make_async_copy(img.at[pl.ds(0, W)],
                                  b.at[pl.ds(0, W)], semr[S]).wait()

    def prep_and_fire(S):
        c0, c8, p0, p8, n0, n8 = rowbufs[S]
        for buf in (c0, p0, n0):
            v = buf[pl.ds(W - 16, 16)]
            plsc.store_scatter(buf, [splat_w], v, mask=mask_last)
        for buf in (c8, p8, n8):
            v = buf[pl.ds(8, 16)]
            plsc.store_scatter(buf, [splat_7], v, mask=mask_first)

        ab = a256[S]

        def a_body(g, carry):
            a = c0[pl.ds(g * 16, 16)]
            ab[pl.ds(g * 16, 16)] = a * 256.0
            return carry
        lax.fori_loop(0, ng_idx, a_body, 0, unroll=2)

        for d, (t, _, rs, dj) in enumerate(_DIRS):
            if dj >= 0:
                buf = {'c': c0, 'p': p0, 'n': n0}[rs]
                off = dj
            else:
                buf = {'c': c8, 'p': p8, 'n': n8}[rs]
                off = 7
            idx_ref = idxs[8 * S + d]

            def i_body(g, carry, buf=buf, off=off, idx_ref=idx_ref):
                av = ab[pl.ds(g * 16, 16)]
                bv = buf[pl.ds(g * 16 + off, 16)]
                idx_ref[pl.ds(g * 16, 16)] = (av + bv).astype(jnp.int32)
                return carry
            lax.fori_loop(0, ng_idx, i_body, 0, unroll=2)
            for c in range(2):
                pltpu.async_copy(tabs[2 * t + c].at[idx_ref],
                                 stages[16 * S + 2 * d + c], semg[S])

    def wait_streams(S):
        for k in range(16):
            pltpu.make_async_copy(tabcat.at[pl.ds(0, W)],
                                  stages[16 * S + k], semg[S]).wait()

    def accum_and_store(r, S):
        outt = outts[S]
        outb = outbs[S]

        def o_body(g, carry):
            sl = pl.ds(g * 16, 16)
            te = to = be = bo = None

            def ext(w, half):
                bits = lax.shift_left(w, sh16) if half == 0 else (w & himask)
                return plsc.bitcast(bits, jnp.float32)

            for d in range(8):
                pm = _PERM[_DIRS[d][1]]
                wt = stages[16 * S + 2 * d][sl]
                wb = stages[16 * S + 2 * d + 1][sl]
                vals = [ext(wt if p < 2 else wb, p % 2) for p in pm]
                te = vals[0] if te is None else te + vals[0]
                to = vals[1] if to is None else to + vals[1]
                be = vals[2] if be is None else be + vals[2]
                bo = vals[3] if bo is None else bo + vals[3]
            sc = sc_even + g * 32
            plsc.store_scatter(outt, [sc], te)
            plsc.store_scatter(outt, [sc + 1], to)
            plsc.store_scatter(outb, [sc], be)
            plsc.store_scatter(outb, [sc + 1], bo)
            return carry
        lax.fori_loop(0, ng_idx, o_body, 0, unroll=2)

        plane = lax.div(r, H)
        orow = r - plane * H
        obase = plane * (2 * H * 2 * W) + orow * (4 * W)
        pltpu.async_copy(outt, out.at[pl.ds(obase, 2 * W)], semo[S])
        pltpu.async_copy(outb, out.at[pl.ds(obase + 2 * W, 2 * W)], semo[S])

    def wait_out(S):
        pltpu.make_async_copy(img.at[pl.ds(0, 2 * W)],
                              outts[S], semo[S]).wait()
        pltpu.make_async_copy(img.at[pl.ds(0, 2 * W)],
                              outbs[S], semo[S]).wait()

    load_rows(row0, 0)
    wait_rows(0)
    prep_and_fire(0)
    load_rows(row0 + 1, 1)

    def pair_body(j, carry):
        r = row0 + 2 * j
        wait_rows(1)
        prep_and_fire(1)
        load_rows(r + 2, 0)
        wait_streams(0)

        @pl.when(j > 0)
        def _():
            wait_out(0)
        accum_and_store(r, 0)
        wait_rows(0)
        prep_and_fire(0)
        load_rows(r + 3, 1)
        wait_streams(1)

        @pl.when(j > 0)
        def _():
            wait_out(1)
        accum_and_store(r + 1, 1)
        return carry

    lax.fori_loop(0, rows_per_worker // 2, pair_body, 0)

    wait_streams(0)
    wait_rows(1)
    wait_out(0)
    wait_out(1)


@functools.partial(jax.jit, static_argnums=(2, 3, 4, 5))
def _sc_call(img_flat, tabcat, nrows, W, H, interpret=False):
    rows_per_worker = nrows // NUM_WORKERS
    mesh = plsc.VectorSubcoreMesh(core_axis_name="c", subcore_axis_name="s",
                                  num_cores=2, num_subcores=16)
    body = functools.partial(_sc_body, H, W, rows_per_worker)
    return pl.kernel(
        body,
        out_type=jax.ShapeDtypeStruct((nrows * 4 * W,), jnp.float32),
        mesh=mesh,
        interpret=interpret,
        compiler_params=pltpu.CompilerParams(needs_layout_passes=False,
                                             use_tc_tiling_on_sc=False),
        scratch_types=dict(
            tabs=[pltpu.VMEM_SHARED((L * L,), jnp.int32) for _ in range(6)],
            rowbufs=[[pltpu.VMEM((W + 16,), jnp.float32) for _ in range(6)]
                     for _ in range(2)],
            idxs=[pltpu.VMEM((W,), jnp.int32) for _ in range(16)],
            stages=[pltpu.VMEM((W,), jnp.int32) for _ in range(32)],
            a256=[pltpu.VMEM((W,), jnp.float32) for _ in range(2)],
            outts=[pltpu.VMEM((2 * W,), jnp.float32) for _ in range(2)],
            outbs=[pltpu.VMEM((2 * W,), jnp.float32) for _ in range(2)],
            semr=[pltpu.SemaphoreType.DMA for _ in range(2)],
            semg=[pltpu.SemaphoreType.DMA for _ in range(2)],
            semo=[pltpu.SemaphoreType.DMA for _ in range(2)],
        ),
    )(img_flat, tabcat)


def kernel(img_lr, h_weight, d_weight, v_weight):
    B, C, H, W = img_lr.shape
    img = img_lr.reshape(-1)

    def pack_pair(lo, hi):
        lob = jax.lax.bitcast_convert_type(
            (0.5 * lo).astype(jnp.bfloat16), jnp.uint16).astype(jnp.uint32)
        hib = jax.lax.bitcast_convert_type(
            (0.5 * hi).astype(jnp.bfloat16), jnp.uint16).astype(jnp.uint32)
        return ((hib << 16) | lob).astype(jnp.int32)

    planes = []
    for w in (h_weight, v_weight, d_weight):
        wf = w.reshape(L * L, 4)
        planes.append(pack_pair(wf[:, 0], wf[:, 1]))
        planes.append(pack_pair(wf[:, 2], wf[:, 3]))
    tabcat = jnp.concatenate(planes)
    out = _sc_call(img, tabcat, B * C * H, W, H)
    return out.reshape(B, C, 2 * H, 2 * W)


# speedup vs baseline: 927.1357x; 1.1225x over previous
import functools

import jax
import jax.numpy as jnp
from jax import lax
from jax.experimental import pallas as pl
from jax.experimental.pallas import tpu as pltpu
from jax.experimental.pallas import tpu_sc as plsc

L = 256
NUM_WORKERS = 32

_PERM = {
    0: (0, 1, 2, 3),
    1: (1, 3, 0, 2),
    2: (3, 2, 1, 0),
    3: (2, 0, 3, 1),
}

_DIRS = (
    (0, 0, 'c', +1),
    (0, 2, 'c', -1),
    (1, 0, 'n', 0),
    (1, 2, 'p', 0),
    (2, 0, 'n', +1),
    (2, 3, 'n', -1),
    (2, 2, 'p', -1),
    (2, 1, 'p', +1),
)


def _sc_body(H, W, rows_per_worker, img, tabcat, out,
             tabs, rowbufs, idxs, stages, outts, outbs,
             semr, semg, semo):
    ng_idx = W // 16

    sid = lax.axis_index("s")

    @pl.when(sid == 0)
    def _stage_tables():
        for k in range(16):
            pltpu.sync_copy(tabcat.at[pl.ds(k * L * L, L * L)], tabs[k])

    plsc.subcore_barrier()

    wid = lax.axis_index("s") * 2 + lax.axis_index("c")
    row0 = wid * rows_per_worker
    rlast = row0 + rows_per_worker - 1

    io16 = lax.iota(jnp.int32, 16)
    mask_last = io16 == 15
    mask_first = io16 == 0
    splat_w = jnp.full((16,), W, jnp.int32)
    splat_7 = jnp.full((16,), 7, jnp.int32)
    sc_even = io16 * 2
    himask = jnp.full((16,), -65536, jnp.int32)
    sh16 = jnp.full((16,), 16, jnp.int32)


    def load_rows(r, S):
        r = jnp.minimum(r, rlast)
        plane = lax.div(r, H)
        orow = r - plane * H
        prev_r = jnp.where(orow == 0, r, r - 1)
        next_r = jnp.where(orow == H - 1, r, r + 1)
        c0, c8, p0, p8, n0, n8 = rowbufs[S]
        for src_r, (b0, b8) in ((r, (c0, c8)), (prev_r, (p0, p8)),
                                (next_r, (n0, n8))):
            pltpu.async_copy(img.at[pl.ds(src_r * W, W)],
                             b0.at[pl.ds(0, W)], semr[S])
            pltpu.async_copy(img.at[pl.ds(src_r * W, W)],
                             b8.at[pl.ds(8, W)], semr[S])

    def wait_rows(S):
        for b in rowbufs[S]:
            pltpu.make_async_copy(img.at[pl.ds(0, W)],
                                  b.at[pl.ds(0, W)], semr[S]).wait()

    def prep_and_fire(S):
        c0, c8, p0, p8, n0, n8 = rowbufs[S]
        for buf in (c0, p0, n0):
            v = buf[pl.ds(W - 16, 16)]
            plsc.store_scatter(buf, [splat_w], v, mask=mask_last)
        for buf in (c8, p8, n8):
            v = buf[pl.ds(8, 16)]
            plsc.store_scatter(buf, [splat_7], v, mask=mask_first)

        bufoff = []
        for d, (t, _, rs, dj) in enumerate(_DIRS):
            if dj >= 0:
                bufoff.append(({'c': c0, 'p': p0, 'n': n0}[rs], dj))
            else:
                bufoff.append(({'c': c8, 'p': p8, 'n': n8}[rs], 7))

        @plsc.parallel_loop(0, ng_idx, unroll=2)
        def i_body(g):
            av = c0[pl.ds(g * 16, 16)] * 256.0
            for d in range(8):
                buf, off = bufoff[d]
                bv = buf[pl.ds(g * 16 + off, 16)]
                idxs[8 * S + d][pl.ds(g * 16, 16)] = \
                    (av + bv).astype(jnp.int32)

        for d in range(8):
            idx_ref = idxs[8 * S + d]
            for c in range(2):
                pltpu.async_copy(tabs[2 * d + c].at[idx_ref],
                                 stages[16 * S + 2 * d + c], semg[S])

    def wait_streams(S):
        for k in range(16):
            pltpu.make_async_copy(tabcat.at[pl.ds(0, W)],
                                  stages[16 * S + k], semg[S]).wait()

    def accum_and_store(r, S):
        outt = outts[S]
        outb = outbs[S]

        @plsc.parallel_loop(0, ng_idx, unroll=2)
        def o_body(g):
            sl = pl.ds(g * 16, 16)
            tacc = bacc = None
            for d in range(8):
                wt = plsc.bitcast(stages[16 * S + 2 * d][sl], jnp.bfloat16)
                wb = plsc.bitcast(stages[16 * S + 2 * d + 1][sl],
                                  jnp.bfloat16)
                tacc = wt if tacc is None else tacc + wt
                bacc = wb if bacc is None else bacc + wb
            tp = plsc.bitcast(tacc, jnp.int32)
            bp = plsc.bitcast(bacc, jnp.int32)
            te = plsc.bitcast(lax.shift_left(tp, sh16), jnp.float32)
            to = plsc.bitcast(tp & himask, jnp.float32)
            be = plsc.bitcast(lax.shift_left(bp, sh16), jnp.float32)
            bo = plsc.bitcast(bp & himask, jnp.float32)
            sc = sc_even + g * 32
            plsc.store_scatter(outt, [sc], te)
            plsc.store_scatter(outt, [sc + 1], to)
            plsc.store_scatter(outb, [sc], be)
            plsc.store_scatter(outb, [sc + 1], bo)

        plane = lax.div(r, H)
        orow = r - plane * H
        obase = plane * (2 * H * 2 * W) + orow * (4 * W)
        pltpu.async_copy(outt, out.at[pl.ds(obase, 2 * W)], semo[S])
        pltpu.async_copy(outb, out.at[pl.ds(obase + 2 * W, 2 * W)], semo[S])

    def wait_out(S):
        pltpu.make_async_copy(img.at[pl.ds(0, 2 * W)],
                              outts[S], semo[S]).wait()
        pltpu.make_async_copy(img.at[pl.ds(0, 2 * W)],
                              outbs[S], semo[S]).wait()

    load_rows(row0, 0)
    wait_rows(0)
    prep_and_fire(0)
    load_rows(row0 + 1, 1)

    def pair_body(j, carry):
        r = row0 + 2 * j
        wait_rows(1)
        prep_and_fire(1)
        load_rows(r + 2, 0)
        wait_streams(0)

        @pl.when(j > 0)
        def _():
            wait_out(0)
        accum_and_store(r, 0)
        wait_rows(0)
        prep_and_fire(0)
        load_rows(r + 3, 1)
        wait_streams(1)

        @pl.when(j > 0)
        def _():
            wait_out(1)
        accum_and_store(r + 1, 1)
        return carry

    lax.fori_loop(0, rows_per_worker // 2, pair_body, 0)

    wait_streams(0)
    wait_rows(1)
    wait_out(0)
    wait_out(1)


@functools.partial(jax.jit, static_argnums=(2, 3, 4, 5))
def _sc_call(img_flat, tabcat, nrows, W, H, interpret=False):
    rows_per_worker = nrows // NUM_WORKERS
    mesh = plsc.VectorSubcoreMesh(core_axis_name="c", subcore_axis_name="s",
                                  num_cores=2, num_subcores=16)
    body = functools.partial(_sc_body, H, W, rows_per_worker)
    return pl.kernel(
        body,
        out_type=jax.ShapeDtypeStruct((nrows * 4 * W,), jnp.float32),
        mesh=mesh,
        interpret=interpret,
        compiler_params=pltpu.CompilerParams(needs_layout_passes=False,
                                             use_tc_tiling_on_sc=False),
        scratch_types=dict(
            tabs=[pltpu.VMEM_SHARED((L * L,), jnp.int32) for _ in range(16)],
            rowbufs=[[pltpu.VMEM((W + 16,), jnp.float32) for _ in range(6)]
                     for _ in range(2)],
            idxs=[pltpu.VMEM((W,), jnp.int32) for _ in range(16)],
            stages=[pltpu.VMEM((W,), jnp.int32) for _ in range(32)],
            outts=[pltpu.VMEM((2 * W,), jnp.float32) for _ in range(2)],
            outbs=[pltpu.VMEM((2 * W,), jnp.float32) for _ in range(2)],
            semr=[pltpu.SemaphoreType.DMA for _ in range(2)],
            semg=[pltpu.SemaphoreType.DMA for _ in range(2)],
            semo=[pltpu.SemaphoreType.DMA for _ in range(2)],
        ),
    )(img_flat, tabcat)


def kernel(img_lr, h_weight, d_weight, v_weight):
    B, C, H, W = img_lr.shape
    img = img_lr.reshape(-1)

    def pack_pair(lo, hi):
        lob = jax.lax.bitcast_convert_type(
            (0.5 * lo).astype(jnp.bfloat16), jnp.uint16).astype(jnp.uint32)
        hib = jax.lax.bitcast_convert_type(
            (0.5 * hi).astype(jnp.bfloat16), jnp.uint16).astype(jnp.uint32)
        return ((hib << 16) | lob).astype(jnp.int32)

    wmap = (h_weight.reshape(L * L, 4), v_weight.reshape(L * L, 4),
            d_weight.reshape(L * L, 4))
    planes = []
    for (t, k, _, _) in _DIRS:
        wf = wmap[t]
        pm = _PERM[k]
        planes.append(pack_pair(wf[:, pm[0]], wf[:, pm[1]]))
        planes.append(pack_pair(wf[:, pm[2]], wf[:, pm[3]]))
    tabcat = jnp.concatenate(planes)
    out = _sc_call(img, tabcat, B * C * H, W, H)
    return out.reshape(B, C, 2 * H, 2 * W)


# speedup vs baseline: 933.2439x; 1.0066x over previous
import functools

import jax
import jax.numpy as jnp
from jax import lax
from jax.experimental import pallas as pl
from jax.experimental.pallas import tpu as pltpu
from jax.experimental.pallas import tpu_sc as plsc

L = 256
NUM_WORKERS = 32

_PERM = {
    0: (0, 1, 2, 3),
    1: (1, 3, 0, 2),
    2: (3, 2, 1, 0),
    3: (2, 0, 3, 1),
}

_DIRS = (
    (0, 0, 'c', +1),
    (0, 2, 'c', -1),
    (1, 0, 'n', 0),
    (1, 2, 'p', 0),
    (2, 0, 'n', +1),
    (2, 3, 'n', -1),
    (2, 2, 'p', -1),
    (2, 1, 'p', +1),
)


def _sc_body(H, W, rows_per_worker, img, tabcat, out,
             tabs, rowbufs, idxs, stages, outtbs, dwf, dwi,
             semr, semg, semo):
    ng_idx = W // 16

    sid = lax.axis_index("s")

    @pl.when(sid == 0)
    def _stage_tables():
        for k in range(8):
            pltpu.sync_copy(tabcat.at[pl.ds(k * 2 * L * L, 2 * L * L)],
                            tabs[k])

    plsc.subcore_barrier()

    wid = lax.axis_index("s") * 2 + lax.axis_index("c")
    row0 = wid * rows_per_worker
    rlast = row0 + rows_per_worker - 1

    io16 = lax.iota(jnp.int32, 16)
    mask_last = io16 == 15
    mask_first = io16 == 0
    splat_w = jnp.full((16,), W, jnp.int32)
    splat_7 = jnp.full((16,), 7, jnp.int32)
    sc_even = io16 * 2
    himask = jnp.full((16,), -65536, jnp.int32)
    sh16 = jnp.full((16,), 16, jnp.int32)


    def load_rows(r, S):
        r = jnp.minimum(r, rlast)
        plane = lax.div(r, H)
        orow = r - plane * H
        prev_r = jnp.where(orow == 0, r, r - 1)
        next_r = jnp.where(orow == H - 1, r, r + 1)
        c0, c8, p0, p8, n0, n8 = rowbufs[S]
        for src_r, (b0, b8) in ((r, (c0, c8)), (prev_r, (p0, p8)),
                                (next_r, (n0, n8))):
            pltpu.async_copy(img.at[pl.ds(src_r * W, W)],
                             b0.at[pl.ds(0, W)], semr[S])
            pltpu.async_copy(img.at[pl.ds(src_r * W, W)],
                             b8.at[pl.ds(8, W)], semr[S])

    def wait_rows(S):
        pltpu.make_async_copy(img.at[pl.ds(0, 6 * W)], dwf, semr[S]).wait()

    def prep_and_fire(S):
        c0, c8, p0, p8, n0, n8 = rowbufs[S]
        for buf in (c0, p0, n0):
            v = buf[pl.ds(W - 16, 16)]
            plsc.store_scatter(buf, [splat_w], v, mask=mask_last)
        for buf in (c8, p8, n8):
            v = buf[pl.ds(8, 16)]
            plsc.store_scatter(buf, [splat_7], v, mask=mask_first)

        bufoff = []
        for d, (t, _, rs, dj) in enumerate(_DIRS):
            if dj >= 0:
                bufoff.append(({'c': c0, 'p': p0, 'n': n0}[rs], dj))
            else:
                bufoff.append(({'c': c8, 'p': p8, 'n': n8}[rs], 7))

        splat_ll = jnp.full((16,), L * L, jnp.int32)

        @plsc.parallel_loop(0, ng_idx, unroll=2)
        def i_body(g):
            av = c0[pl.ds(g * 16, 16)] * 256.0
            for d in range(8):
                buf, off = bufoff[d]
                bv = buf[pl.ds(g * 16 + off, 16)]
                iv = (av + bv).astype(jnp.int32)
                idxs[8 * S + d][pl.ds(g * 16, 16)] = iv
                idxs[8 * S + d][pl.ds(W + g * 16, 16)] = iv + splat_ll

        for d in range(8):
            pltpu.async_copy(tabs[d].at[idxs[8 * S + d]],
                             stages[8 * S + d], semg[S])

    def wait_streams(S):
        pltpu.make_async_copy(tabcat.at[pl.ds(0, 16 * W)], dwi,
                              semg[S]).wait()

    def accum_and_store(r, S):
        outtb = outtbs[S]

        @plsc.parallel_loop(0, ng_idx, unroll=2)
        def o_body(g):
            sl = pl.ds(g * 16, 16)
            slb = pl.ds(W + g * 16, 16)
            tacc = bacc = None
            for d in range(8):
                wt = plsc.bitcast(stages[8 * S + d][sl], jnp.bfloat16)
                wb = plsc.bitcast(stages[8 * S + d][slb], jnp.bfloat16)
                tacc = wt if tacc is None else tacc + wt
                bacc = wb if bacc is None else bacc + wb
            tp = plsc.bitcast(tacc, jnp.int32)
            bp = plsc.bitcast(bacc, jnp.int32)
            te = plsc.bitcast(lax.shift_left(tp, sh16), jnp.float32)
            to = plsc.bitcast(tp & himask, jnp.float32)
            be = plsc.bitcast(lax.shift_left(bp, sh16), jnp.float32)
            bo = plsc.bitcast(bp & himask, jnp.float32)
            sc = sc_even + g * 32
            plsc.store_scatter(outtb, [sc], te)
            plsc.store_scatter(outtb, [sc + 1], to)
            plsc.store_scatter(outtb, [sc + 2 * W], be)
            plsc.store_scatter(outtb, [sc + 2 * W + 1], bo)

        plane = lax.div(r, H)
        orow = r - plane * H
        obase = plane * (2 * H * 2 * W) + orow * (4 * W)
        pltpu.async_copy(outtb, out.at[pl.ds(obase, 4 * W)], semo[S])

    def wait_out(S):
        pltpu.make_async_copy(img.at[pl.ds(0, 4 * W)],
                              outtbs[S], semo[S]).wait()

    load_rows(row0, 0)
    wait_rows(0)
    prep_and_fire(0)
    load_rows(row0 + 1, 1)

    def pair_body(j, carry):
        r = row0 + 2 * j
        wait_rows(1)
        prep_and_fire(1)
        load_rows(r + 2, 0)
        wait_streams(0)

        @pl.when(j > 0)
        def _():
            wait_out(0)
        accum_and_store(r, 0)
        wait_rows(0)
        prep_and_fire(0)
        load_rows(r + 3, 1)
        wait_streams(1)

        @pl.when(j > 0)
        def _():
            wait_out(1)
        accum_and_store(r + 1, 1)
        return carry

    lax.fori_loop(0, rows_per_worker // 2, pair_body, 0)

    wait_streams(0)
    wait_rows(1)
    wait_out(0)
    wait_out(1)


@functools.partial(jax.jit, static_argnums=(2, 3, 4, 5))
def _sc_call(img_flat, tabcat, nrows, W, H, interpret=False):
    rows_per_worker = nrows // NUM_WORKERS
    mesh = plsc.VectorSubcoreMesh(core_axis_name="c", subcore_axis_name="s",
                                  num_cores=2, num_subcores=16)
    body = functools.partial(_sc_body, H, W, rows_per_worker)
    return pl.kernel(
        body,
        out_type=jax.ShapeDtypeStruct((nrows * 4 * W,), jnp.float32),
        mesh=mesh,
        interpret=interpret,
        compiler_params=pltpu.CompilerParams(needs_layout_passes=False,
                                             use_tc_tiling_on_sc=False),
        scratch_types=dict(
            tabs=[pltpu.VMEM_SHARED((2 * L * L,), jnp.int32)
                  for _ in range(8)],
            rowbufs=[[pltpu.VMEM((W + 16,), jnp.float32) for _ in range(6)]
                     for _ in range(2)],
            idxs=[pltpu.VMEM((2 * W,), jnp.int32) for _ in range(16)],
            stages=[pltpu.VMEM((2 * W,), jnp.int32) for _ in range(16)],
            outtbs=[pltpu.VMEM((4 * W,), jnp.float32) for _ in range(2)],
            dwf=pltpu.VMEM((6 * W,), jnp.float32),
            dwi=pltpu.VMEM((16 * W,), jnp.int32),
            semr=[pltpu.SemaphoreType.DMA for _ in range(2)],
            semg=[pltpu.SemaphoreType.DMA for _ in range(2)],
            semo=[pltpu.SemaphoreType.DMA for _ in range(2)],
        ),
    )(img_flat, tabcat)


def kernel(img_lr, h_weight, d_weight, v_weight):
    B, C, H, W = img_lr.shape
    img = img_lr.reshape(-1)

    def pack_pair(lo, hi):
        lob = jax.lax.bitcast_convert_type(
            (0.5 * lo).astype(jnp.bfloat16), jnp.uint16).astype(jnp.uint32)
        hib = jax.lax.bitcast_convert_type(
            (0.5 * hi).astype(jnp.bfloat16), jnp.uint16).astype(jnp.uint32)
        return ((hib << 16) | lob).astype(jnp.int32)

    wmap = (h_weight.reshape(L * L, 4), v_weight.reshape(L * L, 4),
            d_weight.reshape(L * L, 4))
    planes = []
    for (t, k, _, _) in _DIRS:
        wf = wmap[t]
        pm = _PERM[k]
        planes.append(pack_pair(wf[:, pm[0]], wf[:, pm[1]]))
        planes.append(pack_pair(wf[:, pm[2]], wf[:, pm[3]]))
    tabcat = jnp.concatenate(planes)
    out = _sc_call(img, tabcat, B * C * H, W, H)
    return out.reshape(B, C, 2 * H, 2 * W)
